# baseline (speedup 1.0000x reference)
import functools

import jax
import jax.numpy as jnp
from jax import lax
from jax.experimental import pallas as pl
from jax.experimental.pallas import tpu as pltpu
from jax.experimental.pallas import tpu_sc as plsc

_N = 10000
_NP = 10112
_F = 128
_E = 320000
_NW = 32
_EPT = 10240
_EPAD = _EPT * _NW
_RPS = _NP // 16

_mesh = plsc.VectorSubcoreMesh(core_axis_name="c", subcore_axis_name="s")


@functools.partial(
    pl.kernel,
    out_type=(jax.ShapeDtypeStruct((_NW, _NP), jnp.float32),
              jax.ShapeDtypeStruct((_NW, _NP), jnp.float32)),
    mesh=_mesh,
    scratch_types=(
        pltpu.VMEM((_EPT,), jnp.int32),
        pltpu.VMEM((_EPT,), jnp.int32),
        pltpu.VMEM((_NP,), jnp.float32),
        pltpu.VMEM((_NP,), jnp.float32),
    ),
    compiler_params=pltpu.CompilerParams(needs_layout_passes=False),
)
def _degrees(src_hbm, dst_hbm, out_o, out_i, src_v, dst_v, hist_o, hist_i):
    c = lax.axis_index("c")
    s = lax.axis_index("s")
    wid = s * 2 + c

    zero16 = jnp.zeros((16,), jnp.float32)

    def zbody(j, carry):
        hist_o[pl.ds(j * 16, 16)] = zero16
        hist_i[pl.ds(j * 16, 16)] = zero16
        return carry

    lax.fori_loop(0, _NP // 16, zbody, 0)

    pltpu.sync_copy(src_hbm.at[pl.ds(wid * _EPT, _EPT)], src_v)
    pltpu.sync_copy(dst_hbm.at[pl.ds(wid * _EPT, _EPT)], dst_v)

    one16 = jnp.ones((16,), jnp.float32)

    def body(j, carry):
        sl = pl.ds(j * 16, 16)
        plsc.addupdate_scatter(hist_o, [src_v[sl]], one16)
        plsc.addupdate_scatter(hist_i, [dst_v[sl]], one16)
        return carry

    lax.fori_loop(0, _EPT // 16, body, 0)

    pltpu.sync_copy(hist_o, out_o.at[wid])
    pltpu.sync_copy(hist_i, out_i.at[wid])


_KC = 64
_NCHP = 80
_EPP = _NCHP * _KC
_BIGC = 0
_PBIG = 3
_PSML = 1
_EPT_BIG = _PBIG * _EPP
_EPT_SML = _PSML * _EPP
_EBIG = 16 * _EPT_BIG
_NBUF = 4
_GD = 2


@functools.partial(
    pl.kernel,
    out_type=jax.ShapeDtypeStruct((2, _NP, _F), jnp.float32),
    mesh=_mesh,
    scratch_types=(
        pltpu.VMEM((_EPP,), jnp.int32),
        pltpu.VMEM((_NCHP, _KC), jnp.int32),
        pltpu.VMEM((_NBUF, _KC, _F), jnp.float32),
        pltpu.VMEM_SHARED((_NP, _F), jnp.float32),
        pltpu.SemaphoreType.DMA((_NBUF,)),
        pltpu.SemaphoreType.DMA((_NBUF,)),
    ),
)
def _aggregate(hn_hbm, src_hbm, dst2_hbm, out_hbm,
               idx_s, idx_d, rows, acc, gsem, ssem):
    c = lax.axis_index("c")
    s = lax.axis_index("s")
    big = c == _BIGC
    base_e = jnp.where(big, s * _EPT_BIG, _EBIG + s * _EPT_SML)
    base_c = jnp.where(big, s * (_PBIG * _NCHP),
                       (_EBIG // _KC) + s * (_PSML * _NCHP))

    zero16 = jnp.zeros((16,), jnp.float32)

    def zrow(j, carry):
        r = j // (_F // 16)
        l = j % (_F // 16)
        rows[0, r, pl.ds(l * 16, 16)] = zero16
        return carry

    lax.fori_loop(0, _KC * (_F // 16), zrow, 0)
    _NZ = _RPS // _KC
    for t in range(_NZ):
        pltpu.sync_copy(rows.at[0], acc.at[pl.ds(s * _RPS + t * _KC, _KC)])
    pltpu.sync_copy(rows.at[0, :_RPS - _NZ * _KC],
                    acc.at[pl.ds(s * _RPS + _NZ * _KC, _RPS - _NZ * _KC)])
    plsc.subcore_barrier()

    def gather_start(i, b):
        pltpu.async_copy(hn_hbm.at[idx_s.at[pl.ds(i * _KC, _KC)]],
                         rows.at[b], gsem.at[b])

    def gather_wait(i, b):
        pltpu.make_async_copy(hn_hbm.at[idx_s.at[pl.ds(i * _KC, _KC)]],
                              rows.at[b], gsem.at[b]).wait()

    def scatter_start(i, b):
        pltpu.async_copy(rows.at[b], acc.at[idx_d.at[i]], ssem.at[b],
                         add=True)

    def scatter_wait(i, b):
        pltpu.make_async_copy(rows.at[b], acc.at[idx_d.at[i]],
                              ssem.at[b]).wait()

    def run_pass(p):
        pltpu.sync_copy(src_hbm.at[pl.ds(base_e + p * _EPP, _EPP)], idx_s)
        pltpu.sync_copy(dst2_hbm.at[pl.ds(base_c + p * _NCHP, _NCHP)], idx_d)

        for b in range(_GD):
            gather_start(b, b)

        def chunk(i, carry):
            ig = i + _GD

            @pl.when(ig < _NCHP)
            def _():
                b2 = lax.rem(ig, _NBUF)

                @pl.when(ig >= _NBUF)
                def _():
                    scatter_wait(ig - _NBUF, b2)

                gather_start(ig, b2)

            b = lax.rem(i, _NBUF)
            gather_wait(i, b)
            scatter_start(i, b)
            return carry

        lax.fori_loop(0, _NCHP, chunk, 0)

        for d in range(_NBUF):
            j = _NCHP - _NBUF + d
            scatter_wait(j, j % _NBUF)

    run_pass(0)
    for p in range(1, _PBIG):
        @pl.when(big)
        def _():
            run_pass(p)

    plsc.subcore_barrier()
    pltpu.sync_copy(acc.at[pl.ds(s * _RPS, _RPS)],
                    out_hbm.at[c, pl.ds(s * _RPS, _RPS)])


def _norms_body(ho_ref, hi_ref, ns_ref, nd_ref):
    dego = jnp.sum(ho_ref[...], axis=0, keepdims=True)
    degi = jnp.sum(hi_ref[...], axis=0, keepdims=True)
    ns_ref[...] = jnp.where(dego > 0, lax.rsqrt(jnp.maximum(dego, 1.0)), 0.0)
    nd_ref[...] = jnp.where(degi > 0, lax.rsqrt(jnp.maximum(degi, 1.0)), 0.0)


_norms = pl.pallas_call(
    _norms_body,
    out_shape=(jax.ShapeDtypeStruct((1, _NP), jnp.float32),
               jax.ShapeDtypeStruct((1, _NP), jnp.float32)),
)

_R = 2528


def _scale_body(x_ref, n_ref, o_ref):
    o_ref[...] = x_ref[...] * n_ref[...]


_scale = pl.pallas_call(
    _scale_body,
    grid=(_NP // _R,),
    in_specs=[pl.BlockSpec((_R, _F), lambda i: (i, 0)),
              pl.BlockSpec((_R, 1), lambda i: (i, 0))],
    out_specs=pl.BlockSpec((_R, _F), lambda i: (i, 0)),
    out_shape=jax.ShapeDtypeStruct((_NP, _F), jnp.float32),
)


def _mm_relu_body(agg_ref, w_ref, b_ref, nd_ref, ns_ref, o_ref):
    agg = agg_ref[0] + agg_ref[1]
    y = jnp.dot(agg, w_ref[...], preferred_element_type=jnp.float32)
    y = y * nd_ref[...] + b_ref[...]
    o_ref[...] = jnp.maximum(y, 0.0) * ns_ref[...]


_mm_relu = pl.pallas_call(
    _mm_relu_body,
    grid=(_NP // _R,),
    in_specs=[pl.BlockSpec((2, _R, _F), lambda i: (0, i, 0)),
              pl.BlockSpec((_F, _F), lambda i: (0, 0)),
              pl.BlockSpec((1, _F), lambda i: (0, 0)),
              pl.BlockSpec((_R, 1), lambda i: (i, 0)),
              pl.BlockSpec((_R, 1), lambda i: (i, 0))],
    out_specs=pl.BlockSpec((_R, _F), lambda i: (i, 0)),
    out_shape=jax.ShapeDtypeStruct((_NP, _F), jnp.float32),
)


def _mm_out_body(agg_ref, w_ref, b_ref, nd_ref, o_ref):
    agg = agg_ref[0] + agg_ref[1]
    y = jnp.dot(agg, w_ref[...], preferred_element_type=jnp.float32)
    o_ref[...] = y * nd_ref[...] + b_ref[...]


_mm_out = pl.pallas_call(
    _mm_out_body,
    grid=(_NP // _R,),
    in_specs=[pl.BlockSpec((2, _R, _F), lambda i: (0, i, 0)),
              pl.BlockSpec((_F, _F), lambda i: (0, 0)),
              pl.BlockSpec((1, _F), lambda i: (0, 0)),
              pl.BlockSpec((_R, 1), lambda i: (i, 0))],
    out_specs=pl.BlockSpec((_R, _F), lambda i: (i, 0)),
    out_shape=jax.ShapeDtypeStruct((_NP, _F), jnp.float32),
)


def kernel(features, edge_index, W1, b1, W2, b2):
    src = edge_index[0].astype(jnp.int32)
    dst = edge_index[1].astype(jnp.int32)
    pad = jnp.full((_EPAD - _E,), _N, jnp.int32)
    src_p = jnp.concatenate([src, pad])
    dst_p = jnp.concatenate([dst, pad])
    dst2 = dst_p.reshape(_EPAD // _KC, _KC)
    feat_p = jnp.concatenate(
        [features.astype(jnp.float32), jnp.zeros((_NP - _N, _F), jnp.float32)])

    hist_o, hist_i = _degrees(src_p, dst_p)
    ns_row, nd_row = _norms(hist_o, hist_i)
    ns = ns_row.reshape(_NP, 1)
    nd = nd_row.reshape(_NP, 1)

    hn1 = _scale(feat_p, ns)
    agg1 = _aggregate(hn1, src_p, dst2)
    h1n = _mm_relu(agg1, W1, b1.reshape(1, _F), nd, ns)
    agg2 = _aggregate(h1n, src_p, dst2)
    out = _mm_out(agg2, W2, b2.reshape(1, _F), nd)
    return out[:_N]


# speedup vs baseline: 1.1257x; 1.1257x over previous
import functools

import jax
import jax.numpy as jnp
from jax import lax
from jax.experimental import pallas as pl
from jax.experimental.pallas import tpu as pltpu
from jax.experimental.pallas import tpu_sc as plsc

_N = 10000
_NP = 10112
_F = 128
_E = 320000
_NW = 32
_EPT = 10240
_EPAD = _EPT * _NW
_RPS = _NP // 16

_mesh = plsc.VectorSubcoreMesh(core_axis_name="c", subcore_axis_name="s")


@functools.partial(
    pl.kernel,
    out_type=(jax.ShapeDtypeStruct((_NW, _NP), jnp.float32),
              jax.ShapeDtypeStruct((_NW, _NP), jnp.float32)),
    mesh=_mesh,
    scratch_types=(
        pltpu.VMEM((_EPT,), jnp.int32),
        pltpu.VMEM((_EPT,), jnp.int32),
        pltpu.VMEM((_NP,), jnp.float32),
        pltpu.VMEM((_NP,), jnp.float32),
    ),
    compiler_params=pltpu.CompilerParams(needs_layout_passes=False),
)
def _degrees(src_hbm, dst_hbm, out_o, out_i, src_v, dst_v, hist_o, hist_i):
    c = lax.axis_index("c")
    s = lax.axis_index("s")
    wid = s * 2 + c

    zero16 = jnp.zeros((16,), jnp.float32)

    def zbody(j, carry):
        hist_o[pl.ds(j * 16, 16)] = zero16
        hist_i[pl.ds(j * 16, 16)] = zero16
        return carry

    lax.fori_loop(0, _NP // 16, zbody, 0)

    pltpu.sync_copy(src_hbm.at[pl.ds(wid * _EPT, _EPT)], src_v)
    pltpu.sync_copy(dst_hbm.at[pl.ds(wid * _EPT, _EPT)], dst_v)

    one16 = jnp.ones((16,), jnp.float32)

    def body(j, carry):
        sl = pl.ds(j * 16, 16)
        plsc.addupdate_scatter(hist_o, [src_v[sl]], one16)
        plsc.addupdate_scatter(hist_i, [dst_v[sl]], one16)
        return carry

    lax.fori_loop(0, _EPT // 16, body, 0)

    pltpu.sync_copy(hist_o, out_o.at[wid])
    pltpu.sync_copy(hist_i, out_i.at[wid])


_KC = 128
_NCHP = 40
_EPP = _NCHP * _KC
_BIGC = 0
_PBIG = 3
_PSML = 1
_EPT_BIG = _PBIG * _EPP
_EPT_SML = _PSML * _EPP
_EBIG = 16 * _EPT_BIG
_NBUF = 2


@functools.partial(
    pl.kernel,
    out_type=jax.ShapeDtypeStruct((2, _NP, _F), jnp.float32),
    mesh=_mesh,
    scratch_types=(
        pltpu.VMEM((_EPP,), jnp.int32),
        pltpu.VMEM((_NCHP, _KC), jnp.int32),
        pltpu.VMEM((_NBUF, _KC, _F), jnp.float32),
        pltpu.VMEM_SHARED((_NP, _F), jnp.float32),
        pltpu.SemaphoreType.DMA((_NBUF,)),
        pltpu.SemaphoreType.DMA((_NBUF,)),
    ),
)
def _aggregate(hn_hbm, src_hbm, dst2_hbm, out_hbm,
               idx_s, idx_d, rows, acc, gsem, ssem):
    c = lax.axis_index("c")
    s = lax.axis_index("s")
    big = c == _BIGC
    base_e = jnp.where(big, s * _EPT_BIG, _EBIG + s * _EPT_SML)
    base_c = jnp.where(big, s * (_PBIG * _NCHP),
                       (_EBIG // _KC) + s * (_PSML * _NCHP))

    zero16 = jnp.zeros((16,), jnp.float32)

    def zrow(j, carry):
        r = j // (_F // 16)
        l = j % (_F // 16)
        rows[0, r, pl.ds(l * 16, 16)] = zero16
        return carry

    lax.fori_loop(0, _KC * (_F // 16), zrow, 0)
    _NZ = _RPS // _KC
    for t in range(_NZ):
        pltpu.sync_copy(rows.at[0], acc.at[pl.ds(s * _RPS + t * _KC, _KC)])
    pltpu.sync_copy(rows.at[0, :_RPS - _NZ * _KC],
                    acc.at[pl.ds(s * _RPS + _NZ * _KC, _RPS - _NZ * _KC)])
    plsc.subcore_barrier()

    def gather_start(i, b):
        pltpu.async_copy(hn_hbm.at[idx_s.at[pl.ds(i * _KC, _KC)]],
                         rows.at[b], gsem.at[b])

    def gather_wait(i, b):
        pltpu.make_async_copy(hn_hbm.at[idx_s.at[pl.ds(i * _KC, _KC)]],
                              rows.at[b], gsem.at[b]).wait()

    def scatter_start(i, b):
        pltpu.async_copy(rows.at[b], acc.at[idx_d.at[i]], ssem.at[b],
                         add=True)

    def scatter_wait(i, b):
        pltpu.make_async_copy(rows.at[b], acc.at[idx_d.at[i]],
                              ssem.at[b]).wait()

    def run_pass(p):
        pltpu.sync_copy(src_hbm.at[pl.ds(base_e + p * _EPP, _EPP)], idx_s)
        pltpu.sync_copy(dst2_hbm.at[pl.ds(base_c + p * _NCHP, _NCHP)], idx_d)

        gather_start(0, 0)

        def chunk(i, carry):
            b = lax.rem(i, _NBUF)
            gather_wait(i, b)
            scatter_start(i, b)

            @pl.when(i + 1 < _NCHP)
            def _():
                @pl.when(i >= 1)
                def _():
                    scatter_wait(i - 1, 1 - b)

                gather_start(i + 1, 1 - b)

            return carry

        lax.fori_loop(0, _NCHP, chunk, 0)

        scatter_wait(_NCHP - 2, _NCHP % 2)
        scatter_wait(_NCHP - 1, (_NCHP - 1) % 2)

    run_pass(0)
    for p in range(1, _PBIG):
        @pl.when(big)
        def _():
            run_pass(p)

    plsc.subcore_barrier()
    pltpu.sync_copy(acc.at[pl.ds(s * _RPS, _RPS)],
                    out_hbm.at[c, pl.ds(s * _RPS, _RPS)])


def _norms_body(ho_ref, hi_ref, ns_ref, nd_ref):
    dego = jnp.sum(ho_ref[...], axis=0, keepdims=True)
    degi = jnp.sum(hi_ref[...], axis=0, keepdims=True)
    ns_ref[...] = jnp.where(dego > 0, lax.rsqrt(jnp.maximum(dego, 1.0)), 0.0)
    nd_ref[...] = jnp.where(degi > 0, lax.rsqrt(jnp.maximum(degi, 1.0)), 0.0)


_norms = pl.pallas_call(
    _norms_body,
    out_shape=(jax.ShapeDtypeStruct((1, _NP), jnp.float32),
               jax.ShapeDtypeStruct((1, _NP), jnp.float32)),
)

_R = 2528


def _scale_body(x_ref, n_ref, o_ref):
    o_ref[...] = x_ref[...] * n_ref[...]


_scale = pl.pallas_call(
    _scale_body,
    grid=(_NP // _R,),
    in_specs=[pl.BlockSpec((_R, _F), lambda i: (i, 0)),
              pl.BlockSpec((_R, 1), lambda i: (i, 0))],
    out_specs=pl.BlockSpec((_R, _F), lambda i: (i, 0)),
    out_shape=jax.ShapeDtypeStruct((_NP, _F), jnp.float32),
)


def _mm_relu_body(agg_ref, w_ref, b_ref, nd_ref, ns_ref, o_ref):
    agg = agg_ref[0] + agg_ref[1]
    y = jnp.dot(agg, w_ref[...], preferred_element_type=jnp.float32)
    y = y * nd_ref[...] + b_ref[...]
    o_ref[...] = jnp.maximum(y, 0.0) * ns_ref[...]


_mm_relu = pl.pallas_call(
    _mm_relu_body,
    grid=(_NP // _R,),
    in_specs=[pl.BlockSpec((2, _R, _F), lambda i: (0, i, 0)),
              pl.BlockSpec((_F, _F), lambda i: (0, 0)),
              pl.BlockSpec((1, _F), lambda i: (0, 0)),
              pl.BlockSpec((_R, 1), lambda i: (i, 0)),
              pl.BlockSpec((_R, 1), lambda i: (i, 0))],
    out_specs=pl.BlockSpec((_R, _F), lambda i: (i, 0)),
    out_shape=jax.ShapeDtypeStruct((_NP, _F), jnp.float32),
)


def _mm_out_body(agg_ref, w_ref, b_ref, nd_ref, o_ref):
    agg = agg_ref[0] + agg_ref[1]
    y = jnp.dot(agg, w_ref[...], preferred_element_type=jnp.float32)
    o_ref[...] = y * nd_ref[...] + b_ref[...]


_mm_out = pl.pallas_call(
    _mm_out_body,
    grid=(_NP // _R,),
    in_specs=[pl.BlockSpec((2, _R, _F), lambda i: (0, i, 0)),
              pl.BlockSpec((_F, _F), lambda i: (0, 0)),
              pl.BlockSpec((1, _F), lambda i: (0, 0)),
              pl.BlockSpec((_R, 1), lambda i: (i, 0))],
    out_specs=pl.BlockSpec((_R, _F), lambda i: (i, 0)),
    out_shape=jax.ShapeDtypeStruct((_NP, _F), jnp.float32),
)


def kernel(features, edge_index, W1, b1, W2, b2):
    src = edge_index[0].astype(jnp.int32)
    dst = edge_index[1].astype(jnp.int32)
    pad = jnp.full((_EPAD - _E,), _N, jnp.int32)
    src_p = jnp.concatenate([src, pad])
    dst_p = jnp.concatenate([dst, pad])
    dst2 = dst_p.reshape(_EPAD // _KC, _KC)
    feat_p = jnp.concatenate(
        [features.astype(jnp.float32), jnp.zeros((_NP - _N, _F), jnp.float32)])

    hist_o, hist_i = _degrees(src_p, dst_p)
    ns_row, nd_row = _norms(hist_o, hist_i)
    ns = ns_row.reshape(_NP, 1)
    nd = nd_row.reshape(_NP, 1)

    hn1 = _scale(feat_p, ns)
    agg1 = _aggregate(hn1, src_p, dst2)
    h1n = _mm_relu(agg1, W1, b1.reshape(1, _F), nd, ns)
    agg2 = _aggregate(h1n, src_p, dst2)
    out = _mm_out(agg2, W2, b2.reshape(1, _F), nd)
    return out[:_N]
